# baseline (speedup 1.0000x reference)
import functools

import jax
import jax.numpy as jnp
from jax import lax
from jax.experimental import pallas as pl
from jax.experimental.pallas import tpu as pltpu
from jax.experimental.pallas import tpu_sc as plsc

VOCAB = 100000
VOCAB_PAD = 114688
D = 64
C = 2
B = 4096
L = 50

NC, NS = 2, 16
NW = NC * NS
BAGS_PER_TILE = B // NW
GROUPS = BAGS_PER_TILE // 16

PROJ_BLK = 16384


def _proj_body(w_ref, t_ref, o_ref):
    q = jnp.dot(w_ref[...], t_ref[...], preferred_element_type=jnp.float32)
    u0 = lax.bitcast_convert_type(q[0:1, :], jnp.int32)
    u1 = lax.bitcast_convert_type(q[1:2, :], jnp.int32)
    r0 = (u0 + 0x7FFF + ((u0 >> 16) & 1)) & jnp.int32(-65536)
    r1 = ((u1 + 0x7FFF + ((u1 >> 16) & 1)) >> 16) & jnp.int32(0xFFFF)
    o_ref[0:1, :] = lax.bitcast_convert_type(r0 | r1, jnp.float32)


def _project(table_t, w_pad):
    return pl.pallas_call(
        _proj_body,
        grid=(VOCAB_PAD // PROJ_BLK,),
        in_specs=[
            pl.BlockSpec((C, D), lambda i: (0, 0)),
            pl.BlockSpec((D, PROJ_BLK), lambda i: (0, i)),
        ],
        out_specs=pl.BlockSpec((8, PROJ_BLK), lambda i: (0, i)),
        out_shape=jax.ShapeDtypeStruct((8, VOCAB_PAD), jnp.float32),
    )(w_pad, table_t)


def _sc_body(pt_hbm, idx_hbm, bias_hbm, out_hbm,
             ptab_sh, ptab_v, idxt_v, bias_v, out_v, sem):
    sid = lax.axis_index("s")
    wid = sid * NC + lax.axis_index("c")
    base = wid * BAGS_PER_TILE

    @pl.when(sid == 0)
    def _():
        pltpu.sync_copy(pt_hbm.at[0], ptab_sh)

    pltpu.sync_copy(idx_hbm.at[:, pl.ds(base, BAGS_PER_TILE)], idxt_v)
    pltpu.sync_copy(bias_hbm, bias_v)
    plsc.subcore_barrier()
    pltpu.sync_copy(ptab_sh, ptab_v)

    b0 = bias_v[0]
    b1 = bias_v[1]
    lanes = lax.iota(jnp.int32, 16)

    for g in range(GROUPS):
        def body(l, accs):
            a0, a1 = accs
            ids = idxt_v[l, pl.ds(16 * g, 16)]
            u = plsc.bitcast(plsc.load_gather(ptab_v, [ids]), jnp.int32)
            hi = plsc.bitcast(u & jnp.int32(-65536), jnp.float32)
            lo = plsc.bitcast(u << 16, jnp.float32)
            return (a0 + hi, a1 + lo)

        acc0, acc1 = lax.fori_loop(
            0, L, body,
            (jnp.zeros((16,), jnp.float32), jnp.zeros((16,), jnp.float32)),
            unroll=10)
        c0 = acc0 * (1.0 / L) + b0
        c1 = acc1 * (1.0 / L) + b1
        pos = lanes * 2 + (32 * g)
        plsc.store_scatter(out_v, [pos], c0)
        plsc.store_scatter(out_v, [pos + 1], c1)

    pltpu.sync_copy(out_v, out_hbm.at[pl.ds(base * C, BAGS_PER_TILE * C)])


@jax.jit
def _run(text, table, fc_w, fc_b):
    bias2 = jnp.repeat(fc_b, 16).reshape(C, 16)
    packed = _project(table.T, fc_w)
    idx_t = text.T

    mesh = plsc.VectorSubcoreMesh(core_axis_name="c", subcore_axis_name="s",
                                  num_cores=NC, num_subcores=NS)
    out_flat = pl.kernel(
        _sc_body,
        out_type=jax.ShapeDtypeStruct((B * C,), jnp.float32),
        mesh=mesh,
        compiler_params=pltpu.CompilerParams(use_tc_tiling_on_sc=True,
                                             needs_layout_passes=False,
                                             skip_device_barrier=True),
        scratch_types=[
            pltpu.VMEM_SHARED((VOCAB_PAD,), jnp.float32),
            pltpu.VMEM((VOCAB_PAD,), jnp.float32),
            pltpu.VMEM((L, BAGS_PER_TILE), jnp.int32),
            pltpu.VMEM((C, 16), jnp.float32),
            pltpu.VMEM((BAGS_PER_TILE * C,), jnp.float32),
            pltpu.SemaphoreType.DMA,
        ],
    )(packed, idx_t, bias2)
    return out_flat.reshape(B, C)


def kernel(text, table, fc_w, fc_b):
    return _run(text, table, fc_w, fc_b)


# speedup vs baseline: 1.0224x; 1.0224x over previous
import functools

import jax
import jax.numpy as jnp
from jax import lax
from jax.experimental import pallas as pl
from jax.experimental.pallas import tpu as pltpu
from jax.experimental.pallas import tpu_sc as plsc

VOCAB = 100000
VOCAB_PAD = 114688
D = 64
C = 2
B = 4096
L = 50

NC, NS = 2, 16
NW = NC * NS
BAGS_PER_TILE = B // NW
GROUPS = BAGS_PER_TILE // 16

PROJ_BLK = 16384


def _proj_body(b_ref, w_ref, t_ref, o_ref):
    q = jnp.dot(w_ref[...], t_ref[...], preferred_element_type=jnp.float32)
    u0 = lax.bitcast_convert_type(q[0:1, :] + b_ref[0], jnp.int32)
    u1 = lax.bitcast_convert_type(q[1:2, :] + b_ref[1], jnp.int32)
    r0 = (u0 + 0x7FFF + ((u0 >> 16) & 1)) & jnp.int32(-65536)
    r1 = ((u1 + 0x7FFF + ((u1 >> 16) & 1)) >> 16) & jnp.int32(0xFFFF)
    o_ref[0:1, :] = lax.bitcast_convert_type(r0 | r1, jnp.float32)


def _project(table_t, w, b):
    return pl.pallas_call(
        _proj_body,
        grid=(VOCAB_PAD // PROJ_BLK,),
        in_specs=[
            pl.BlockSpec(memory_space=pltpu.SMEM),
            pl.BlockSpec((C, D), lambda i: (0, 0)),
            pl.BlockSpec((D, PROJ_BLK), lambda i: (0, i)),
        ],
        out_specs=pl.BlockSpec((8, PROJ_BLK), lambda i: (0, i)),
        out_shape=jax.ShapeDtypeStruct((8, VOCAB_PAD), jnp.float32),
    )(b, w, table_t)


HALF = VOCAB_PAD // 2


def _sc_body(pt_hbm, idx_hbm, out_hbm,
             ptab_sh, ptab_v, idxt_v, out_v, sem):
    sid = lax.axis_index("s")
    wid = sid * NC + lax.axis_index("c")
    base = wid * BAGS_PER_TILE

    @pl.when(sid == 0)
    def _():
        pltpu.sync_copy(pt_hbm.at[0, pl.ds(0, HALF)], ptab_sh.at[pl.ds(0, HALF)])

    @pl.when(sid == 1)
    def _():
        pltpu.sync_copy(pt_hbm.at[0, pl.ds(HALF, HALF)],
                        ptab_sh.at[pl.ds(HALF, HALF)])

    pltpu.sync_copy(idx_hbm.at[:, pl.ds(base, BAGS_PER_TILE)], idxt_v)
    plsc.subcore_barrier()
    pltpu.sync_copy(ptab_sh, ptab_v)

    lanes = lax.iota(jnp.int32, 16)

    for g in range(GROUPS):
        def body(l, accs):
            a0, a1 = accs
            ids = idxt_v[l, pl.ds(16 * g, 16)]
            u = plsc.bitcast(plsc.load_gather(ptab_v, [ids]), jnp.int32)
            hi = plsc.bitcast(u & jnp.int32(-65536), jnp.float32)
            lo = plsc.bitcast(u << 16, jnp.float32)
            return (a0 + hi, a1 + lo)

        acc0, acc1 = lax.fori_loop(
            0, L, body,
            (jnp.zeros((16,), jnp.float32), jnp.zeros((16,), jnp.float32)),
            unroll=10)
        c0 = acc0 * (1.0 / L)
        c1 = acc1 * (1.0 / L)
        pos = lanes * 2 + (32 * g)
        plsc.store_scatter(out_v, [pos], c0)
        plsc.store_scatter(out_v, [pos + 1], c1)

    pltpu.sync_copy(out_v, out_hbm.at[pl.ds(base * C, BAGS_PER_TILE * C)])


@jax.jit
def _run(text, table, fc_w, fc_b):
    packed = _project(table.T, fc_w, fc_b)
    idx_t = text.T

    mesh = plsc.VectorSubcoreMesh(core_axis_name="c", subcore_axis_name="s",
                                  num_cores=NC, num_subcores=NS)
    out_flat = pl.kernel(
        _sc_body,
        out_type=jax.ShapeDtypeStruct((B * C,), jnp.float32),
        mesh=mesh,
        compiler_params=pltpu.CompilerParams(use_tc_tiling_on_sc=True,
                                             needs_layout_passes=False),
        scratch_types=[
            pltpu.VMEM_SHARED((VOCAB_PAD,), jnp.float32),
            pltpu.VMEM((VOCAB_PAD,), jnp.float32),
            pltpu.VMEM((L, BAGS_PER_TILE), jnp.int32),
            pltpu.VMEM((BAGS_PER_TILE * C,), jnp.float32),
            pltpu.SemaphoreType.DMA,
        ],
    )(packed, idx_t)
    return out_flat.reshape(B, C)


def kernel(text, table, fc_w, fc_b):
    return _run(text, table, fc_w, fc_b)


# speedup vs baseline: 1.0232x; 1.0008x over previous
import jax
import jax.numpy as jnp
from jax import lax
from jax.experimental import pallas as pl
from jax.experimental.pallas import tpu as pltpu
from jax.experimental.pallas import tpu_sc as plsc

VOCAB = 100000
VOCAB_PAD = 114688
D = 64
C = 2
B = 4096
L = 50

NC, NS = 2, 16
NW = NC * NS
BAGS_PER_TILE = B // NW
GROUPS = BAGS_PER_TILE // 16

PROJ_BLK = 16384


def _proj_body(b_ref, w_ref, t_ref, o_ref):
    q = jnp.dot(w_ref[...], t_ref[...], preferred_element_type=jnp.float32)
    u0 = lax.bitcast_convert_type(q[0:1, :] + b_ref[0], jnp.int32)
    u1 = lax.bitcast_convert_type(q[1:2, :] + b_ref[1], jnp.int32)
    r0 = (u0 + 0x7FFF + ((u0 >> 16) & 1)) & jnp.int32(-65536)
    r1 = ((u1 + 0x7FFF + ((u1 >> 16) & 1)) >> 16) & jnp.int32(0xFFFF)
    o_ref[0:1, :] = lax.bitcast_convert_type(r0 | r1, jnp.float32)


def _project(table_t, w, b):
    return pl.pallas_call(
        _proj_body,
        grid=(VOCAB_PAD // PROJ_BLK,),
        in_specs=[
            pl.BlockSpec(memory_space=pltpu.SMEM),
            pl.BlockSpec((C, D), lambda i: (0, 0)),
            pl.BlockSpec((D, PROJ_BLK), lambda i: (0, i)),
        ],
        out_specs=pl.BlockSpec((8, PROJ_BLK), lambda i: (0, i)),
        out_shape=jax.ShapeDtypeStruct((8, VOCAB_PAD), jnp.float32),
    )(b, w, table_t)


HALF = VOCAB_PAD // 2


def _sc_body(pt_hbm, idx_hbm, out_hbm,
             ptab_sh, ptab_v, idxt_v, out_v, sem):
    sid = lax.axis_index("s")
    wid = sid * NC + lax.axis_index("c")
    base = wid * BAGS_PER_TILE

    @pl.when(sid == 0)
    def _():
        pltpu.sync_copy(pt_hbm.at[0, pl.ds(0, HALF)], ptab_sh.at[pl.ds(0, HALF)])

    @pl.when(sid == 1)
    def _():
        pltpu.sync_copy(pt_hbm.at[0, pl.ds(HALF, HALF)],
                        ptab_sh.at[pl.ds(HALF, HALF)])

    pltpu.sync_copy(idx_hbm.at[:, pl.ds(base, BAGS_PER_TILE)], idxt_v)
    plsc.subcore_barrier()
    pltpu.sync_copy(ptab_sh, ptab_v)

    lanes = lax.iota(jnp.int32, 16)

    for g in range(GROUPS):
        def body(l, accs):
            a0, a1 = accs
            ids = idxt_v[l, pl.ds(16 * g, 16)]
            u = plsc.bitcast(plsc.load_gather(ptab_v, [ids]), jnp.int32)
            hi = plsc.bitcast(u & jnp.int32(-65536), jnp.float32)
            lo = plsc.bitcast(u << 16, jnp.float32)
            return (a0 + hi, a1 + lo)

        acc0, acc1 = lax.fori_loop(
            0, L, body,
            (jnp.zeros((16,), jnp.float32), jnp.zeros((16,), jnp.float32)),
            unroll=10)
        c0 = acc0 * (1.0 / L)
        c1 = acc1 * (1.0 / L)
        pos = lanes * 2 + (32 * g)
        plsc.store_scatter(out_v, [pos], c0)
        plsc.store_scatter(out_v, [pos + 1], c1)

    pltpu.sync_copy(out_v, out_hbm.at[pl.ds(base * C, BAGS_PER_TILE * C)])


@jax.jit
def _run(text, table, fc_w, fc_b):
    packed = _project(table.T, fc_w, fc_b)
    idx_t = text.T

    mesh = plsc.VectorSubcoreMesh(core_axis_name="c", subcore_axis_name="s",
                                  num_cores=NC, num_subcores=NS)
    out_flat = pl.kernel(
        _sc_body,
        out_type=jax.ShapeDtypeStruct((B * C,), jnp.float32),
        mesh=mesh,
        compiler_params=pltpu.CompilerParams(use_tc_tiling_on_sc=True,
                                             needs_layout_passes=False),
        scratch_types=[
            pltpu.VMEM_SHARED((VOCAB_PAD,), jnp.float32),
            pltpu.VMEM((VOCAB_PAD,), jnp.float32),
            pltpu.VMEM((L, BAGS_PER_TILE), jnp.int32),
            pltpu.VMEM((BAGS_PER_TILE * C,), jnp.float32),
            pltpu.SemaphoreType.DMA,
        ],
    )(packed, idx_t)
    return out_flat.reshape(B, C)


def kernel(text, table, fc_w, fc_b):
    return _run(text, table, fc_w, fc_b)
